# baseline (speedup 1.0000x reference)
import functools

import jax
import jax.numpy as jnp
from jax import lax
from jax.experimental import pallas as pl
from jax.experimental.pallas import tpu as pltpu
from jax.experimental.pallas import tpu_sc as plsc

_B, _S, _V = 2, 2048, 32000
_N = _B * _S
_NW = 16
_RPW = _N // _NW
_CHUNK = 128
_NCH = _RPW // _CHUNK
_L = 16


def _sc_loss(pred_flat, tgt, msk):
    mesh = plsc.VectorSubcoreMesh(
        core_axis_name="c", subcore_axis_name="s", num_cores=1)

    @functools.partial(
        pl.kernel,
        out_type=jax.ShapeDtypeStruct((_L,), jnp.float32),
        mesh=mesh,
        compiler_params=pltpu.CompilerParams(needs_layout_passes=False),
        scratch_types=[
            pltpu.VMEM((_RPW,), jnp.int32),
            pltpu.VMEM((_RPW,), jnp.float32),
            pltpu.VMEM((_NCH, _CHUNK), jnp.int32),
            pltpu.VMEM((_NCH, _CHUNK), jnp.float32),
            pltpu.VMEM((_L, _L), jnp.float32),
            pltpu.VMEM((_L, _L), jnp.float32),
            pltpu.VMEM((_L,), jnp.float32),
            pltpu.VMEM_SHARED((_L, _L), jnp.float32),
            pltpu.SemaphoreType.DMA,
        ],
    )
    def k(pred_hbm, tgt_hbm, msk_hbm, out_hbm,
          tgt_v, msk_v, idx_v, gat_v, loc_v, tot_v, tmp_v, shacc, sem):
        wid = lax.axis_index("s")
        base = wid * _RPW
        pltpu.sync_copy(tgt_hbm.at[pl.ds(base, _RPW)], tgt_v)
        pltpu.sync_copy(msk_hbm.at[pl.ds(base, _RPW)], msk_v)

        for g in range(_RPW // _L):
            t16 = tgt_v[pl.ds(g * _L, _L)]
            rows = (base + g * _L) + lax.iota(jnp.int32, _L)
            idx_v[g // (_CHUNK // _L), pl.ds((g % (_CHUNK // _L)) * _L, _L)] = (
                rows * _V + t16)

        for r in range(_L):
            loc_v[r, pl.ds(0, _L)] = jnp.zeros((_L,), jnp.float32)

        @pl.when(wid == 0)
        def _():
            pltpu.sync_copy(loc_v, shacc)

        cps = [pltpu.async_copy(pred_hbm.at[idx_v.at[j]], gat_v.at[j], sem)
               for j in range(_NCH)]
        for cp in cps:
            cp.wait()

        gacc = jnp.zeros((_L,), jnp.float32)
        macc = jnp.zeros((_L,), jnp.float32)
        for g in range(_RPW // _L):
            j, o = g // (_CHUNK // _L), (g % (_CHUNK // _L)) * _L
            gv = gat_v[j, pl.ds(o, _L)]
            mv = msk_v[pl.ds(g * _L, _L)]
            gacc = gacc + gv * mv
            macc = macc + mv
        loc_v[0, pl.ds(0, _L)] = gacc
        loc_v[1, pl.ds(0, _L)] = macc

        plsc.subcore_barrier()
        pltpu.sync_copy(loc_v, shacc.at[lax.iota(jnp.int32, _L)], add=True)
        plsc.subcore_barrier()

        pltpu.sync_copy(shacc, tot_v)
        tg = tot_v[0, pl.ds(0, _L)]
        tm = tot_v[1, pl.ds(0, _L)]
        lane = lax.iota(jnp.int32, _L)
        for shift in (8, 4, 2, 1):
            tmp_v[...] = tg
            tg = tg + plsc.load_gather(tmp_v, [lane ^ shift])
            tmp_v[...] = tm
            tm = tm + plsc.load_gather(tmp_v, [lane ^ shift])
        tmp_v[...] = -tg / tm

        @pl.when(wid == 0)
        def _():
            pltpu.sync_copy(tmp_v, out_hbm)

    return k(pred_flat, tgt, msk)


def kernel(predicts, targets, masks):
    pred_flat = predicts.reshape(-1)
    tgt = targets.reshape(-1).astype(jnp.int32)
    msk = masks.reshape(-1)
    out = _sc_loss(pred_flat, tgt, msk)
    return out[0]


# speedup vs baseline: 9.7972x; 9.7972x over previous
import functools

import jax
import jax.numpy as jnp
from jax import lax
from jax.experimental import pallas as pl
from jax.experimental.pallas import tpu as pltpu
from jax.experimental.pallas import tpu_sc as plsc

_B, _S, _V = 2, 2048, 32000
_N = _B * _S
_NW = 16
_RPW = _N // _NW
_CHUNK = 128
_NCH = _RPW // _CHUNK
_L = 16

_TILE_W = 8 * 128
_TROW_W = (_V // 128) * _TILE_W
_SLAB_W = _S * _V


def _sc_loss(pred, tgt, msk):
    mesh = plsc.VectorSubcoreMesh(
        core_axis_name="c", subcore_axis_name="s", num_cores=1)

    @functools.partial(
        pl.kernel,
        out_type=jax.ShapeDtypeStruct((_L,), jnp.float32),
        mesh=mesh,
        compiler_params=pltpu.CompilerParams(
            needs_layout_passes=False, use_tc_tiling_on_sc=True),
        scratch_types=[
            pltpu.VMEM((_RPW,), jnp.int32),
            pltpu.VMEM((_RPW,), jnp.float32),
            pltpu.VMEM((_L * 8, 128), jnp.float32),
            pltpu.VMEM((_L * 8, 128), jnp.float32),
            pltpu.VMEM((_L, _L), jnp.float32),
            pltpu.VMEM((_L, _L), jnp.float32),
            pltpu.VMEM((_L,), jnp.float32),
            pltpu.VMEM_SHARED((_L, _L), jnp.float32),
            pltpu.SemaphoreType.DMA,
            pltpu.SemaphoreType.DMA,
        ],
    )
    def k(pred_hbm, tgt_hbm, msk_hbm, out_hbm,
          tgt_v, msk_v, bufa, bufb, loc_v, tot_v, tmp_v, shacc, sema, semb):
        wid = lax.axis_index("s")
        base = wid * _RPW
        pltpu.sync_copy(tgt_hbm.at[pl.ds(base, _RPW)], tgt_v)
        pltpu.sync_copy(msk_hbm.at[pl.ds(base, _RPW)], msk_v)

        for r in range(_L):
            loc_v[r, pl.ds(0, _L)] = jnp.zeros((_L,), jnp.float32)

        @pl.when(wid == 0)
        def _():
            pltpu.sync_copy(loc_v, shacc)

        bufs = (bufa, bufb)
        sems = (sema, semb)
        ngrp = _RPW // _L

        def fire(g):
            buf, sem = bufs[g % 2], sems[g % 2]
            tvec = tgt_v[pl.ds(g * _L, _L)]
            b = (base + g * _L) >> 11
            s_grp = (base + g * _L) & (_S - 1)
            for j in range(_L):
                s0 = pl.multiple_of(s_grp + (j & -8), 8)
                t0 = pl.multiple_of(tvec[j] & -128, 128)
                pltpu.async_copy(pred_hbm.at[b, pl.ds(s0, 8), pl.ds(t0, 128)],
                                 buf.at[pl.ds(j * 8, 8)], sem)

        def drain(g):
            buf, sem = bufs[g % 2], sems[g % 2]
            pltpu.make_async_copy(
                pred_hbm.at[0, pl.ds(0, _L * 8), pl.ds(0, 128)], buf,
                sem).wait()

        gacc = jnp.zeros((_L,), jnp.float32)
        macc = jnp.zeros((_L,), jnp.float32)
        lane = lax.iota(jnp.int32, _L)
        rows16 = lane * 8 + (lane & 7)

        def extract(g, gacc, macc):
            buf = bufs[g % 2]
            lanes16 = tgt_v[pl.ds(g * _L, _L)] & 127
            gv = plsc.load_gather(buf, [rows16, lanes16])
            mv = msk_v[pl.ds(g * _L, _L)]
            return gacc + gv * mv, macc + mv

        fire(0)
        for g in range(1, ngrp):
            fire(g)
            drain(g - 1)
            gacc, macc = extract(g - 1, gacc, macc)
        drain(ngrp - 1)
        gacc, macc = extract(ngrp - 1, gacc, macc)
        loc_v[0, pl.ds(0, _L)] = gacc
        loc_v[1, pl.ds(0, _L)] = macc

        plsc.subcore_barrier()
        pltpu.sync_copy(loc_v, shacc.at[lax.iota(jnp.int32, _L)], add=True)
        plsc.subcore_barrier()

        pltpu.sync_copy(shacc, tot_v)
        tg = tot_v[0, pl.ds(0, _L)]
        tm = tot_v[1, pl.ds(0, _L)]
        for shift in (8, 4, 2, 1):
            tmp_v[...] = tg
            tg = tg + plsc.load_gather(tmp_v, [lane ^ shift])
            tmp_v[...] = tm
            tm = tm + plsc.load_gather(tmp_v, [lane ^ shift])
        tmp_v[...] = -tg / tm

        @pl.when(wid == 0)
        def _():
            pltpu.sync_copy(tmp_v, out_hbm)

    return k(pred, tgt, msk)


def kernel(predicts, targets, masks):
    tgt = targets.reshape(-1).astype(jnp.int32)
    msk = masks.reshape(-1)
    out = _sc_loss(predicts, tgt, msk)
    return out[0]


# speedup vs baseline: 11.0239x; 1.1252x over previous
import functools

import jax
import jax.numpy as jnp
from jax import lax
from jax.experimental import pallas as pl
from jax.experimental.pallas import tpu as pltpu
from jax.experimental.pallas import tpu_sc as plsc

_B, _S, _V = 2, 2048, 32000
_N = _B * _S
_NC = 2
_NW = 16 * _NC
_RPW = _N // _NW
_L = 16
_NGRP = _RPW // _L
_NBUF = 4


def _sc_loss(pred, tgt, msk):
    mesh = plsc.VectorSubcoreMesh(core_axis_name="c", subcore_axis_name="s")

    @functools.partial(
        pl.kernel,
        out_type=jax.ShapeDtypeStruct((_NC, 2, _L), jnp.float32),
        mesh=mesh,
        compiler_params=pltpu.CompilerParams(
            needs_layout_passes=False, use_tc_tiling_on_sc=True),
        scratch_types=[
            pltpu.VMEM((_RPW,), jnp.int32),
            pltpu.VMEM((_RPW,), jnp.float32),
            pltpu.VMEM((_L * 8, 128), jnp.float32),
            pltpu.VMEM((_L * 8, 128), jnp.float32),
            pltpu.VMEM((_L * 8, 128), jnp.float32),
            pltpu.VMEM((_L * 8, 128), jnp.float32),
            pltpu.VMEM((_L, _L), jnp.float32),
            pltpu.VMEM((_L, _L), jnp.float32),
            pltpu.VMEM((_L,), jnp.float32),
            pltpu.VMEM_SHARED((_L, _L), jnp.float32),
            pltpu.SemaphoreType.DMA,
            pltpu.SemaphoreType.DMA,
            pltpu.SemaphoreType.DMA,
            pltpu.SemaphoreType.DMA,
        ],
    )
    def k(pred_hbm, tgt_hbm, msk_hbm, out_hbm,
          tgt_v, msk_v, buf0, buf1, buf2, buf3, loc_v, tot_v, tmp_v, shacc,
          sem0, sem1, sem2, sem3):
        cid = lax.axis_index("c")
        sid = lax.axis_index("s")
        wid = sid * _NC + cid
        base = wid * _RPW
        pltpu.sync_copy(tgt_hbm.at[pl.ds(base, _RPW)], tgt_v)
        pltpu.sync_copy(msk_hbm.at[pl.ds(base, _RPW)], msk_v)

        for r in range(_L):
            loc_v[r, pl.ds(0, _L)] = jnp.zeros((_L,), jnp.float32)

        @pl.when(sid == 0)
        def _():
            pltpu.sync_copy(loc_v, shacc)

        bufs = (buf0, buf1, buf2, buf3)
        sems = (sem0, sem1, sem2, sem3)

        def fire(g):
            buf, sem = bufs[g % _NBUF], sems[g % _NBUF]
            tvec = tgt_v[pl.ds(g * _L, _L)]
            b = (base + g * _L) >> 11
            s_grp = (base + g * _L) & (_S - 1)
            for j in range(_L):
                s0 = pl.multiple_of(s_grp + (j & -8), 8)
                t0 = pl.multiple_of(tvec[j] & -128, 128)
                pltpu.async_copy(pred_hbm.at[b, pl.ds(s0, 8), pl.ds(t0, 128)],
                                 buf.at[pl.ds(j * 8, 8)], sem)

        def drain(g):
            buf, sem = bufs[g % _NBUF], sems[g % _NBUF]
            pltpu.make_async_copy(
                pred_hbm.at[0, pl.ds(0, _L * 8), pl.ds(0, 128)], buf,
                sem).wait()

        gacc = jnp.zeros((_L,), jnp.float32)
        macc = jnp.zeros((_L,), jnp.float32)
        lane = lax.iota(jnp.int32, _L)
        rows16 = lane * 8 + (lane & 7)

        def extract(g, gacc, macc):
            buf = bufs[g % _NBUF]
            lanes16 = tgt_v[pl.ds(g * _L, _L)] & 127
            gv = plsc.load_gather(buf, [rows16, lanes16])
            mv = msk_v[pl.ds(g * _L, _L)]
            return gacc + gv * mv, macc + mv

        for g in range(_NBUF - 1):
            fire(g)
        for g in range(_NBUF - 1, _NGRP):
            fire(g)
            drain(g - _NBUF + 1)
            gacc, macc = extract(g - _NBUF + 1, gacc, macc)
        for g in range(_NGRP - _NBUF + 1, _NGRP):
            drain(g)
            gacc, macc = extract(g, gacc, macc)

        loc_v[0, pl.ds(0, _L)] = gacc
        loc_v[1, pl.ds(0, _L)] = macc

        plsc.subcore_barrier()
        pltpu.sync_copy(loc_v, shacc.at[lax.iota(jnp.int32, _L)], add=True)
        plsc.subcore_barrier()

        pltpu.sync_copy(shacc, tot_v)
        tg = tot_v[0, pl.ds(0, _L)]
        tm = tot_v[1, pl.ds(0, _L)]
        for shift in (8, 4, 2, 1):
            tmp_v[...] = tg
            tg = tg + plsc.load_gather(tmp_v, [lane ^ shift])
            tmp_v[...] = tm
            tm = tm + plsc.load_gather(tmp_v, [lane ^ shift])
        loc_v[0, pl.ds(0, _L)] = tg
        loc_v[1, pl.ds(0, _L)] = tm

        @pl.when(sid == 0)
        def _():
            pltpu.sync_copy(loc_v.at[pl.ds(0, 2)], out_hbm.at[cid])

    return k(pred, tgt, msk)


def kernel(predicts, targets, masks):
    tgt = targets.reshape(-1).astype(jnp.int32)
    msk = masks.reshape(-1)
    out = _sc_loss(predicts, tgt, msk)
    g = out[0, 0, 0] + out[1, 0, 0]
    m = out[0, 1, 0] + out[1, 1, 0]
    return -g / m


# speedup vs baseline: 11.0992x; 1.0068x over previous
import functools

import jax
import jax.numpy as jnp
from jax import lax
from jax.experimental import pallas as pl
from jax.experimental.pallas import tpu as pltpu
from jax.experimental.pallas import tpu_sc as plsc

_B, _S, _V = 2, 2048, 32000
_N = _B * _S
_NC = 2
_NW = 16 * _NC
_RPW = _N // _NW
_L = 16
_NGRP = _RPW // _L
_NBUF = 6


def _sc_loss(pred, tgt, msk):
    mesh = plsc.VectorSubcoreMesh(core_axis_name="c", subcore_axis_name="s")

    @functools.partial(
        pl.kernel,
        out_type=jax.ShapeDtypeStruct((_NC, 2, _L), jnp.float32),
        mesh=mesh,
        compiler_params=pltpu.CompilerParams(
            needs_layout_passes=False, use_tc_tiling_on_sc=True),
        scratch_types=[
            pltpu.VMEM((_RPW,), jnp.int32),
            pltpu.VMEM((_RPW,), jnp.float32),
            pltpu.VMEM((_L * 8, 128), jnp.float32),
            pltpu.VMEM((_L * 8, 128), jnp.float32),
            pltpu.VMEM((_L * 8, 128), jnp.float32),
            pltpu.VMEM((_L * 8, 128), jnp.float32),
            pltpu.VMEM((_L * 8, 128), jnp.float32),
            pltpu.VMEM((_L * 8, 128), jnp.float32),
            pltpu.VMEM((_L, _L), jnp.float32),
            pltpu.VMEM((_L, _L), jnp.float32),
            pltpu.VMEM((_L,), jnp.float32),
            pltpu.VMEM_SHARED((_L, _L), jnp.float32),
            pltpu.SemaphoreType.DMA,
            pltpu.SemaphoreType.DMA,
            pltpu.SemaphoreType.DMA,
            pltpu.SemaphoreType.DMA,
            pltpu.SemaphoreType.DMA,
            pltpu.SemaphoreType.DMA,
        ],
    )
    def k(pred_hbm, tgt_hbm, msk_hbm, out_hbm,
          tgt_v, msk_v, buf0, buf1, buf2, buf3, buf4, buf5, loc_v, tot_v,
          tmp_v, shacc, sem0, sem1, sem2, sem3, sem4, sem5):
        cid = lax.axis_index("c")
        sid = lax.axis_index("s")
        wid = sid * _NC + cid
        base = wid * _RPW
        pltpu.sync_copy(tgt_hbm.at[pl.ds(base, _RPW)], tgt_v)
        pltpu.sync_copy(msk_hbm.at[pl.ds(base, _RPW)], msk_v)

        for r in range(_L):
            loc_v[r, pl.ds(0, _L)] = jnp.zeros((_L,), jnp.float32)

        @pl.when(sid == 0)
        def _():
            pltpu.sync_copy(loc_v, shacc)

        bufs = (buf0, buf1, buf2, buf3, buf4, buf5)
        sems = (sem0, sem1, sem2, sem3, sem4, sem5)

        def fire(g):
            buf, sem = bufs[g % _NBUF], sems[g % _NBUF]
            tvec = tgt_v[pl.ds(g * _L, _L)]
            b = (base + g * _L) >> 11
            s_grp = (base + g * _L) & (_S - 1)
            for j in range(_L):
                s0 = pl.multiple_of(s_grp + (j & -8), 8)
                t0 = pl.multiple_of(tvec[j] & -128, 128)
                pltpu.async_copy(pred_hbm.at[b, pl.ds(s0, 8), pl.ds(t0, 128)],
                                 buf.at[pl.ds(j * 8, 8)], sem)

        def drain(g):
            buf, sem = bufs[g % _NBUF], sems[g % _NBUF]
            pltpu.make_async_copy(
                pred_hbm.at[0, pl.ds(0, _L * 8), pl.ds(0, 128)], buf,
                sem).wait()

        gacc = jnp.zeros((_L,), jnp.float32)
        macc = jnp.zeros((_L,), jnp.float32)
        lane = lax.iota(jnp.int32, _L)
        rows16 = lane * 8 + (lane & 7)

        def extract(g, gacc, macc):
            buf = bufs[g % _NBUF]
            lanes16 = tgt_v[pl.ds(g * _L, _L)] & 127
            gv = plsc.load_gather(buf, [rows16, lanes16])
            mv = msk_v[pl.ds(g * _L, _L)]
            return gacc + gv * mv, macc + mv

        for g in range(_NBUF - 1):
            fire(g)
        for g in range(_NBUF - 1, _NGRP):
            fire(g)
            drain(g - _NBUF + 1)
            gacc, macc = extract(g - _NBUF + 1, gacc, macc)
        for g in range(_NGRP - _NBUF + 1, _NGRP):
            drain(g)
            gacc, macc = extract(g, gacc, macc)

        loc_v[0, pl.ds(0, _L)] = gacc
        loc_v[1, pl.ds(0, _L)] = macc

        plsc.subcore_barrier()
        pltpu.sync_copy(loc_v, shacc.at[lax.iota(jnp.int32, _L)], add=True)
        plsc.subcore_barrier()

        pltpu.sync_copy(shacc, tot_v)
        tg = tot_v[0, pl.ds(0, _L)]
        tm = tot_v[1, pl.ds(0, _L)]
        for shift in (8, 4, 2, 1):
            tmp_v[...] = tg
            tg = tg + plsc.load_gather(tmp_v, [lane ^ shift])
            tmp_v[...] = tm
            tm = tm + plsc.load_gather(tmp_v, [lane ^ shift])
        loc_v[0, pl.ds(0, _L)] = tg
        loc_v[1, pl.ds(0, _L)] = tm

        @pl.when(sid == 0)
        def _():
            pltpu.sync_copy(loc_v.at[pl.ds(0, 2)], out_hbm.at[cid])

    return k(pred, tgt, msk)


def kernel(predicts, targets, masks):
    tgt = targets.reshape(-1).astype(jnp.int32)
    msk = masks.reshape(-1)
    out = _sc_loss(predicts, tgt, msk)
    g = out[0, 0, 0] + out[1, 0, 0]
    m = out[0, 1, 0] + out[1, 1, 0]
    return -g / m
